# baseline (speedup 1.0000x reference)
import functools

import jax
import jax.numpy as jnp
from jax import lax
from jax.experimental import pallas as pl
from jax.experimental.pallas import tpu as pltpu
from jax.experimental.pallas import tpu_sc as plsc

_N = 8192
_RANK = _N // 2 - 1
_ROWS = 64 * 32
_NW = 32
_RPW = _ROWS // _NW
_CHUNKS = _N // 16
_INT_MIN = -(2**31)


def _scan_hist(hist_v, rank_rem):
    def body(j, st):
        bidx, below, carry = st
        v16 = hist_v[pl.ds(j * 16, 16)]
        cum = plsc.cumsum(v16) + carry
        le = (cum <= rank_rem).astype(jnp.int32)
        return (
            bidx + jnp.sum(le),
            below + jnp.sum(v16 * le),
            carry + jnp.sum(v16),
        )

    z = jnp.int32(0)
    bidx, below, _ = lax.fori_loop(0, 16, body, (z, z, z))
    return bidx, below


def _sc_body(x_hbm, out_hbm, u_v, hist_v, res_v):
    int_min = jnp.int32(_INT_MIN)
    wid = lax.axis_index("s") * 2 + lax.axis_index("c")
    lane0 = lax.iota(jnp.int32, 16) == 0
    ones = jnp.ones((16,), jnp.int32)

    def row_body(r, _):
        row = wid * _RPW + r
        pltpu.sync_copy(x_hbm.at[pl.ds(row * _N, _N)], u_v)

        for j in range(16):
            hist_v[pl.ds(j * 16, 16)] = jnp.zeros((16,), jnp.int32)

        def conv_body(c, _):
            v = u_v[pl.ds(c * 16, 16)]
            skey = jnp.where(v >= 0, v, int_min - v)
            u = skey ^ int_min
            u_v[pl.ds(c * 16, 16)] = u
            plsc.addupdate_scatter(hist_v, [lax.shift_right_logical(u, 24)], ones)
            return 0

        lax.fori_loop(0, _CHUNKS, conv_body, 0)

        rank_rem = jnp.int32(_RANK)
        digit, below = _scan_hist(hist_v, rank_rem)
        prefix = digit
        rank_rem = rank_rem - below

        for p in range(1, 4):
            shift = 24 - 8 * p
            pfx_b = jnp.full((16,), prefix, jnp.int32)

            for j in range(16):
                hist_v[pl.ds(j * 16, 16)] = jnp.zeros((16,), jnp.int32)

            def hist_body(c, _, shift=shift, pfx_b=pfx_b):
                u = u_v[pl.ds(c * 16, 16)]
                match = lax.shift_right_logical(u, shift + 8) == pfx_b
                bucket = lax.shift_right_logical(u, shift) & 255
                plsc.addupdate_scatter(hist_v, [bucket], ones, mask=match)
                return 0

            lax.fori_loop(0, _CHUNKS, hist_body, 0)

            digit, below = _scan_hist(hist_v, rank_rem)
            prefix = lax.shift_left(prefix, 8) | digit
            rank_rem = rank_rem - below

        skey_ans = prefix ^ int_min
        vbits = jnp.where(skey_ans >= 0, skey_ans, int_min - skey_ans)
        ans = plsc.bitcast(jnp.full((16,), vbits, jnp.int32), jnp.float32)
        plsc.store_scatter(res_v, [jnp.full((16,), r, jnp.int32)], ans, mask=lane0)
        return 0

    lax.fori_loop(0, _RPW, row_body, 0)
    pltpu.sync_copy(res_v, out_hbm.at[pl.ds(wid * _RPW, _RPW)])


_sc_kernel = functools.partial(
    pl.kernel,
    out_type=jax.ShapeDtypeStruct((_ROWS,), jnp.float32),
    mesh=plsc.VectorSubcoreMesh(core_axis_name="c", subcore_axis_name="s"),
    compiler_params=pltpu.CompilerParams(needs_layout_passes=False),
    scratch_types=[
        pltpu.VMEM((_N,), jnp.int32),
        pltpu.VMEM((256,), jnp.int32),
        pltpu.VMEM((_RPW,), jnp.float32),
    ],
)(_sc_body)


def kernel(x):
    b0, b1, n = x.shape
    xi = jax.lax.bitcast_convert_type(x, jnp.int32).reshape(b0 * b1 * n)
    out = _sc_kernel(xi)
    return out.reshape(b0, b1)


# speedup vs baseline: 1.2098x; 1.2098x over previous
import functools

import jax
import jax.numpy as jnp
from jax import lax
from jax.experimental import pallas as pl
from jax.experimental.pallas import tpu as pltpu
from jax.experimental.pallas import tpu_sc as plsc

_N = 8192
_RANK = _N // 2 - 1
_ROWS = 64 * 32
_NW = 32
_RPW = _ROWS // _NW
_CHUNKS = _N // 16
_UNROLL = 16
_INT_MIN = -(2**31)
_BIG = 2**30


def _scan256(hist_v, rank):
    sums = [jnp.sum(hist_v[pl.ds(j * 16, 16)]) for j in range(16)]
    carry = [jnp.int32(0)]
    for j in range(16):
        carry.append(carry[j] + sums[j])
    jstar = jnp.int32(0)
    carry_js = jnp.int32(0)
    for j in range(1, 16):
        le = carry[j] <= rank
        jstar = jstar + le.astype(jnp.int32)
        carry_js = jnp.maximum(carry_js, jnp.where(le, carry[j], 0))
    v = hist_v[pl.ds(jstar * 16, 16)]
    cum = plsc.cumsum(v) + carry_js
    le = cum <= rank
    inner = jnp.sum(le.astype(jnp.int32))
    digit = jstar * 16 + inner
    below = jnp.maximum(jnp.max(jnp.where(le, cum, 0)), carry_js)
    above = jnp.min(jnp.where(le, jnp.int32(_BIG), cum))
    return digit, below, above


def _sc_body(x_hbm, out_hbm, u_v, hist_v, res_v, sem):
    wid = lax.axis_index("s") * 2 + lax.axis_index("c")
    lane0 = lax.iota(jnp.int32, 16) == 0
    ones = jnp.ones((16,), jnp.int32)
    c24 = jnp.full((16,), 24, jnp.int32)
    c31 = jnp.full((16,), 31, jnp.int32)
    m255 = jnp.full((16,), 255, jnp.int32)
    row0 = wid * _RPW

    def dma(r):
        return pltpu.make_async_copy(
            x_hbm.at[pl.ds((row0 + r) * _N, _N)],
            u_v.at[pl.ds((r & 1) * _N, _N)],
            sem,
        )

    dma(0).start()

    def clear_hist():
        for j in range(16):
            hist_v[pl.ds(j * 16, 16)] = jnp.zeros((16,), jnp.int32)

    def row_body(r, _):
        base = (r & 1) * _N
        dma(r).wait()

        @pl.when(r + 1 < _RPW)
        def _():
            dma(r + 1).start()

        clear_hist()

        def p0(i, _):
            for k in range(_UNROLL):
                off = base + (i * _UNROLL + k) * 16
                b = plsc.bitcast(u_v[pl.ds(off, 16)], jnp.int32)
                t = lax.shift_right_logical(b, c24)
                s = lax.shift_right_arithmetic(b, c31)
                o = t ^ ((s & 127) | 128)
                plsc.addupdate_scatter(hist_v, [o], ones)
            return 0

        lax.fori_loop(0, _CHUNKS // _UNROLL, p0, 0)

        rank = jnp.int32(_RANK)
        o_digit, below, above = _scan256(hist_v, rank)
        rank = rank - below
        neg = o_digit < 128
        pfx = jnp.where(neg, o_digit ^ 255, o_digit ^ 128)
        rank = jnp.where(neg, (above - below) - 1 - rank, rank)

        for shift in (16, 8, 0):
            clear_hist()
            xpfx = lax.shift_left(pfx, shift + 8) ^ jnp.int32(_INT_MIN)
            xpfx_b = jnp.full((16,), xpfx, jnp.int32)
            thr = jnp.int32(_INT_MIN + (1 << (shift + 8)))
            sh_b = jnp.full((16,), shift, jnp.int32)

            def pp(i, _, xpfx_b=xpfx_b, thr=thr, sh_b=sh_b):
                for k in range(_UNROLL):
                    off = base + (i * _UNROLL + k) * 16
                    b = plsc.bitcast(u_v[pl.ds(off, 16)], jnp.int32)
                    match = (b ^ xpfx_b) < thr
                    bucket = lax.shift_right_logical(b, sh_b) & m255
                    plsc.addupdate_scatter(hist_v, [bucket], ones, mask=match)
                return 0

            lax.fori_loop(0, _CHUNKS // _UNROLL, pp, 0)

            digit, below, _above = _scan256(hist_v, rank)
            pfx = lax.shift_left(pfx, 8) | digit
            rank = rank - below

        ans = plsc.bitcast(jnp.full((16,), pfx, jnp.int32), jnp.float32)
        plsc.store_scatter(res_v, [jnp.full((16,), r, jnp.int32)], ans, mask=lane0)
        return 0

    lax.fori_loop(0, _RPW, row_body, 0)
    pltpu.sync_copy(res_v, out_hbm.at[pl.ds(row0, _RPW)])


_sc_kernel = functools.partial(
    pl.kernel,
    out_type=jax.ShapeDtypeStruct((_ROWS,), jnp.float32),
    mesh=plsc.VectorSubcoreMesh(core_axis_name="c", subcore_axis_name="s"),
    compiler_params=pltpu.CompilerParams(needs_layout_passes=False),
    scratch_types=[
        pltpu.VMEM((2 * _N,), jnp.float32),
        pltpu.VMEM((256,), jnp.int32),
        pltpu.VMEM((_RPW,), jnp.float32),
        pltpu.SemaphoreType.DMA,
    ],
)(_sc_body)


def kernel(x):
    b0, b1, n = x.shape
    out = _sc_kernel(x.reshape(b0 * b1 * n))
    return out.reshape(b0, b1)


# speedup vs baseline: 3.6235x; 2.9952x over previous
import functools

import jax
import jax.numpy as jnp
from jax import lax
from jax.experimental import pallas as pl
from jax.experimental.pallas import tpu as pltpu
from jax.experimental.pallas import tpu_sc as plsc

_N = 8192
_RANK = _N // 2 - 1
_ROWS = 64 * 32
_NW = 32
_RPW = _ROWS // _NW
_CHUNKS = _N // 16
_UNROLL = 16
_INT_MIN = -(2**31)
_BIG = 2**30


def _scan256(hist_v, rank):
    sums = [jnp.sum(hist_v[pl.ds(j * 16, 16)]) for j in range(16)]
    carry = [jnp.int32(0)]
    for j in range(16):
        carry.append(carry[j] + sums[j])
    jstar = jnp.int32(0)
    carry_js = jnp.int32(0)
    for j in range(1, 16):
        le = carry[j] <= rank
        jstar = jstar + le.astype(jnp.int32)
        carry_js = jnp.maximum(carry_js, jnp.where(le, carry[j], 0))
    v = hist_v[pl.ds(jstar * 16, 16)]
    cum = plsc.cumsum(v) + carry_js
    le = cum <= rank
    inner = jnp.sum(le.astype(jnp.int32))
    digit = jstar * 16 + inner
    below = jnp.maximum(jnp.max(jnp.where(le, cum, 0)), carry_js)
    above = jnp.min(jnp.where(le, jnp.int32(_BIG), cum))
    return digit, below, above


def _sc_body(x_hbm, out_hbm, u_v, hist_v, res_v, sem):
    wid = lax.axis_index("s") * 2 + lax.axis_index("c")
    lane0 = lax.iota(jnp.int32, 16) == 0
    ones = jnp.ones((16,), jnp.int32)
    c24 = jnp.full((16,), 24, jnp.int32)
    c31 = jnp.full((16,), 31, jnp.int32)
    m255 = jnp.full((16,), 255, jnp.int32)
    row0 = wid * _RPW

    def dma(r):
        return pltpu.make_async_copy(
            x_hbm.at[pl.ds((row0 + r) * _N, _N)],
            u_v.at[pl.ds((r & 1) * _N, _N)],
            sem,
        )

    dma(0).start()

    def clear_hist():
        for j in range(16):
            hist_v[pl.ds(j * 16, 16)] = jnp.zeros((16,), jnp.int32)

    def row_body(r, _):
        base = (r & 1) * _N
        dma(r).wait()

        @pl.when(r + 1 < _RPW)
        def _():
            dma(r + 1).start()

        clear_hist()

        @plsc.parallel_loop(0, _CHUNKS, 1, unroll=_UNROLL)
        def _(c):
            b = plsc.bitcast(u_v[pl.ds(base + c * 16, 16)], jnp.int32)
            t = lax.shift_right_logical(b, c24)
            s = lax.shift_right_arithmetic(b, c31)
            o = t ^ ((s & 127) | 128)
            plsc.addupdate_scatter(hist_v, [o], ones)

        rank = jnp.int32(_RANK)
        o_digit, below, above = _scan256(hist_v, rank)
        rank = rank - below
        neg = o_digit < 128
        pfx = jnp.where(neg, o_digit ^ 255, o_digit ^ 128)
        rank = jnp.where(neg, (above - below) - 1 - rank, rank)

        for shift in (16, 8, 0):
            clear_hist()
            xpfx = lax.shift_left(pfx, shift + 8) ^ jnp.int32(_INT_MIN)
            xpfx_b = jnp.full((16,), xpfx, jnp.int32)
            thr = jnp.int32(_INT_MIN + (1 << (shift + 8)))
            sh_b = jnp.full((16,), shift, jnp.int32)

            @plsc.parallel_loop(0, _CHUNKS, 1, unroll=_UNROLL)
            def _(c, xpfx_b=xpfx_b, thr=thr, sh_b=sh_b):
                b = plsc.bitcast(u_v[pl.ds(base + c * 16, 16)], jnp.int32)
                match = (b ^ xpfx_b) < thr
                bucket = lax.shift_right_logical(b, sh_b) & m255
                plsc.addupdate_scatter(hist_v, [bucket], ones, mask=match)

            digit, below, _above = _scan256(hist_v, rank)
            pfx = lax.shift_left(pfx, 8) | digit
            rank = rank - below

        ans = plsc.bitcast(jnp.full((16,), pfx, jnp.int32), jnp.float32)
        plsc.store_scatter(res_v, [jnp.full((16,), r, jnp.int32)], ans, mask=lane0)
        return 0

    lax.fori_loop(0, _RPW, row_body, 0)
    pltpu.sync_copy(res_v, out_hbm.at[pl.ds(row0, _RPW)])


_sc_kernel = functools.partial(
    pl.kernel,
    out_type=jax.ShapeDtypeStruct((_ROWS,), jnp.float32),
    mesh=plsc.VectorSubcoreMesh(core_axis_name="c", subcore_axis_name="s"),
    compiler_params=pltpu.CompilerParams(needs_layout_passes=False),
    scratch_types=[
        pltpu.VMEM((2 * _N,), jnp.float32),
        pltpu.VMEM((256,), jnp.int32),
        pltpu.VMEM((_RPW,), jnp.float32),
        pltpu.SemaphoreType.DMA,
    ],
)(_sc_body)


def kernel(x):
    b0, b1, n = x.shape
    out = _sc_kernel(x.reshape(b0 * b1 * n))
    return out.reshape(b0, b1)


# speedup vs baseline: 6.0277x; 1.6635x over previous
import functools

import jax
import jax.numpy as jnp
from jax import lax
from jax.experimental import pallas as pl
from jax.experimental.pallas import tpu as pltpu
from jax.experimental.pallas import tpu_sc as plsc

_N = 8192
_RANK = _N // 2 - 1
_ROWS = 64 * 32
_NW = 32
_SC_ROWS = 1024
_TC_BLOCK = 128
_CHUNKS = _N // 16
_UNROLL = 16
_INT_MIN = -(2**31)
_BIG = 2**30


def _scan256(hist_v, rank):
    sums = [jnp.sum(hist_v[pl.ds(j * 16, 16)]) for j in range(16)]
    carry = [jnp.int32(0)]
    for j in range(16):
        carry.append(carry[j] + sums[j])
    jstar = jnp.int32(0)
    carry_js = jnp.int32(0)
    for j in range(1, 16):
        le = carry[j] <= rank
        jstar = jstar + le.astype(jnp.int32)
        carry_js = jnp.maximum(carry_js, jnp.where(le, carry[j], 0))
    v = hist_v[pl.ds(jstar * 16, 16)]
    cum = plsc.cumsum(v) + carry_js
    le = cum <= rank
    inner = jnp.sum(le.astype(jnp.int32))
    digit = jstar * 16 + inner
    below = jnp.maximum(jnp.max(jnp.where(le, cum, 0)), carry_js)
    above = jnp.min(jnp.where(le, jnp.int32(_BIG), cum))
    return digit, below, above


def _sc_body(x_hbm, out_hbm, u_v, hist_v, res_v, sem):
    rpw = _SC_ROWS // _NW
    wid = lax.axis_index("s") * 2 + lax.axis_index("c")
    lane0 = lax.iota(jnp.int32, 16) == 0
    ones = jnp.ones((16,), jnp.int32)
    c24 = jnp.full((16,), 24, jnp.int32)
    c31 = jnp.full((16,), 31, jnp.int32)
    m255 = jnp.full((16,), 255, jnp.int32)
    row0 = wid * rpw

    def dma(r):
        return pltpu.make_async_copy(
            x_hbm.at[pl.ds((row0 + r) * _N, _N)],
            u_v.at[pl.ds((r & 1) * _N, _N)],
            sem,
        )

    dma(0).start()

    def clear_hist():
        for j in range(16):
            hist_v[pl.ds(j * 16, 16)] = jnp.zeros((16,), jnp.int32)

    def row_body(r, _):
        base = (r & 1) * _N
        dma(r).wait()

        @pl.when(r + 1 < rpw)
        def _():
            dma(r + 1).start()

        clear_hist()

        @plsc.parallel_loop(0, _CHUNKS, 1, unroll=_UNROLL)
        def _(c):
            b = plsc.bitcast(u_v[pl.ds(base + c * 16, 16)], jnp.int32)
            t = lax.shift_right_logical(b, c24)
            s = lax.shift_right_arithmetic(b, c31)
            o = t ^ ((s & 127) | 128)
            plsc.addupdate_scatter(hist_v, [o], ones)

        rank = jnp.int32(_RANK)
        o_digit, below, above = _scan256(hist_v, rank)
        rank = rank - below
        neg = o_digit < 128
        pfx = jnp.where(neg, o_digit ^ 255, o_digit ^ 128)
        rank = jnp.where(neg, (above - below) - 1 - rank, rank)

        for shift in (16, 8, 0):
            clear_hist()
            xpfx = lax.shift_left(pfx, shift + 8) ^ jnp.int32(_INT_MIN)
            xpfx_b = jnp.full((16,), xpfx, jnp.int32)
            thr = jnp.int32(_INT_MIN + (1 << (shift + 8)))
            sh_b = jnp.full((16,), shift, jnp.int32)

            @plsc.parallel_loop(0, _CHUNKS, 1, unroll=_UNROLL)
            def _(c, xpfx_b=xpfx_b, thr=thr, sh_b=sh_b):
                b = plsc.bitcast(u_v[pl.ds(base + c * 16, 16)], jnp.int32)
                match = (b ^ xpfx_b) < thr
                bucket = lax.shift_right_logical(b, sh_b) & m255
                plsc.addupdate_scatter(hist_v, [bucket], ones, mask=match)

            digit, below, _above = _scan256(hist_v, rank)
            pfx = lax.shift_left(pfx, 8) | digit
            rank = rank - below

        ans = plsc.bitcast(jnp.full((16,), pfx, jnp.int32), jnp.float32)
        plsc.store_scatter(res_v, [jnp.full((16,), r, jnp.int32)], ans, mask=lane0)
        return 0

    lax.fori_loop(0, rpw, row_body, 0)
    pltpu.sync_copy(res_v, out_hbm.at[pl.ds(row0, rpw)])


_sc_kernel = functools.partial(
    pl.kernel,
    out_type=jax.ShapeDtypeStruct((_SC_ROWS,), jnp.float32),
    mesh=plsc.VectorSubcoreMesh(core_axis_name="c", subcore_axis_name="s"),
    compiler_params=pltpu.CompilerParams(needs_layout_passes=False),
    scratch_types=[
        pltpu.VMEM((2 * _N,), jnp.float32),
        pltpu.VMEM((256,), jnp.int32),
        pltpu.VMEM((_SC_ROWS // _NW,), jnp.float32),
        pltpu.SemaphoreType.DMA,
    ],
)(_sc_body)


def _tc_body(x_ref, o_ref):
    x = x_ref[...]
    v = jax.lax.bitcast_convert_type(x, jnp.int32)
    int_min = jnp.int32(_INT_MIN)
    skey = jnp.where(v >= 0, v, int_min - v)
    rows = x.shape[0]
    acc = jnp.zeros((rows, 1), jnp.int32)
    for bit in range(31, -1, -1):
        mask_val = jnp.int32(_INT_MIN if bit == 31 else 1 << bit)
        cand = acc | mask_val
        thr = cand ^ int_min
        cnt = jnp.sum((skey < thr).astype(jnp.int32), axis=1, keepdims=True)
        acc = jnp.where(cnt <= _RANK, cand, acc)
    skey_ans = acc ^ int_min
    vbits = jnp.where(skey_ans >= 0, skey_ans, int_min - skey_ans)
    o_ref[...] = jax.lax.bitcast_convert_type(vbits, jnp.float32)


def _tc_kernel(xr):
    rows = xr.shape[0]
    return pl.pallas_call(
        _tc_body,
        grid=(rows // _TC_BLOCK,),
        in_specs=[pl.BlockSpec((_TC_BLOCK, _N), lambda i: (i, 0))],
        out_specs=pl.BlockSpec((_TC_BLOCK, 1), lambda i: (i, 0)),
        out_shape=jax.ShapeDtypeStruct((rows, 1), jnp.float32),
    )(xr)


def kernel(x):
    b0, b1, n = x.shape
    xr = x.reshape(b0 * b1, n)
    out_sc = _sc_kernel(xr[:_SC_ROWS].reshape(-1))
    out_tc = _tc_kernel(xr[_SC_ROWS:])
    out = jnp.concatenate([out_sc, out_tc[:, 0]])
    return out.reshape(b0, b1)
